# baseline (speedup 1.0000x reference)
import functools

import jax
import jax.numpy as jnp
import numpy as np
from jax import lax
from jax.experimental import pallas as pl
from jax.experimental.pallas import tpu as pltpu
from jax.experimental.pallas import tpu_sc as plsc

_B, _L, _D_IN = 4, 2048, 768
_D_KEY, _D_VAL = 128, 768
_N_BITS = 16
_N = _B * _L

_SC_NC = 2
_SC_NS = 16
_SC_NW = _SC_NC * _SC_NS
_PARTS = 4
_PN = _N // _PARTS
_TOK_PER_W = _PN // _SC_NW
_SC_CHUNK = 32
_SC_STEPS = _TOK_PER_W // _SC_CHUNK

_HEAD_TB = 1024
_TAIL_TB = 512


def _pf45_table() -> np.ndarray:
    r = np.arange(45, dtype=np.float64)
    a = 2.0 * np.pi * r
    cols = np.stack([
        np.cos(a), np.cos(a / 3.0), np.cos(a / 9.0),
        np.sin(a), np.sin(a / 3.0), np.sin(a / 9.0),
        (r % 5 == 0).astype(np.float64), (r % 5 == 1).astype(np.float64),
    ], axis=1)
    return np.tanh(cols).astype(np.float32)


def _head_body(h_ref, t_ref, wk_ref, bk_ref, wpk_ref, wpp_ref, bp_ref,
               proj_ref, pf45_ref, keys_ref, idx_ref):
    h = h_ref[...]
    t = t_ref[...]
    k1 = jnp.dot(h, wk_ref[...], preferred_element_type=jnp.float32) + bk_ref[...]
    r45 = t - 45.0 * jnp.floor(t / 45.0)
    oh = (lax.broadcasted_iota(jnp.int32, (_HEAD_TB, 45), 1)
          == r45.astype(jnp.int32)).astype(jnp.float32)
    tbl = jnp.dot(pf45_ref[...], wpp_ref[...],
                  preferred_element_type=jnp.float32)
    keys = (jnp.dot(k1, wpk_ref[...], preferred_element_type=jnp.float32)
            + jnp.dot(oh, tbl, preferred_element_type=jnp.float32)
            + bp_ref[...])
    keys_ref[...] = keys
    logits = jnp.dot(keys, proj_ref[...], preferred_element_type=jnp.float32)
    w = (jnp.int32(1) << jnp.arange(_N_BITS, dtype=jnp.int32))[None, :]
    idx = jnp.sum((logits > 0.0).astype(jnp.int32) * w, axis=1, keepdims=True)
    idx_ref[...] = idx.reshape(_HEAD_TB // 128, 128)


def _tail_body(h_ref, p_ref, keys_ref, ksel_ref, w1a_ref, w1b_ref, w1c_ref,
               b1_ref, w2_ref, b2_ref, gin_ref, bin_ref, gpr_ref, bpr_ref,
               y_ref, asum_ref, csum_ref):
    def ln(x, g, b):
        m = jnp.mean(x, axis=1, keepdims=True)
        v = jnp.mean((x - m) ** 2, axis=1, keepdims=True)
        return (x - m) / jnp.sqrt(v + 1e-5) * g + b

    h = h_ref[...]
    p = p_ref[...]
    lnh = ln(h, gin_ref[...], bin_ref[...])
    lnp = ln(p, gpr_ref[...], bpr_ref[...])
    conf = jax.nn.sigmoid(
        jnp.sum(keys_ref[...] * ksel_ref[...], axis=1, keepdims=True)
        / jnp.sqrt(jnp.float32(_D_KEY)))
    m1 = (jnp.dot(lnh, w1a_ref[...], preferred_element_type=jnp.float32)
          + jnp.dot(lnp, w1b_ref[...], preferred_element_type=jnp.float32)
          + conf * w1c_ref[0:1, :] + b1_ref[...])
    s = m1 * jax.nn.sigmoid(m1)
    pre = jnp.dot(s, w2_ref[...], preferred_element_type=jnp.float32) + b2_ref[...]
    alpha = jnp.clip(jax.nn.sigmoid(pre), 0.0, 1.0)
    y_ref[...] = (1.0 - alpha) * h + alpha * (h + p)

    @pl.when(pl.program_id(0) == 0)
    def _():
        asum_ref[...] = jnp.zeros_like(asum_ref)
        csum_ref[...] = jnp.zeros_like(csum_ref)

    asum_ref[...] += jnp.sum(alpha).reshape(1, 1)
    csum_ref[...] += jnp.sum(conf).reshape(1, 1)


def _tail_body_alias(h_ref, p_ref, keys_ref, ksel_ref, w1a_ref, w1b_ref,
                     w1c_ref, b1_ref, w2_ref, b2_ref, gin_ref, bin_ref,
                     gpr_ref, bpr_ref, y0_ref, y_ref, asum_ref, csum_ref):
    del y0_ref
    _tail_body(h_ref, p_ref, keys_ref, ksel_ref, w1a_ref, w1b_ref, w1c_ref,
               b1_ref, w2_ref, b2_ref, gin_ref, bin_ref, gpr_ref, bpr_ref,
               y_ref, asum_ref, csum_ref)


def _sc_gather_body(idx_hbm, vtab_hbm, ktab_hbm, pred_hbm, ksel_hbm,
                    idx_v, vrows0, vrows1, krows0, krows1,
                    sem_v0, sem_v1, sem_k0, sem_k1):
    wid = lax.axis_index("s") * _SC_NC + lax.axis_index("c")
    base = wid * _TOK_PER_W
    pltpu.sync_copy(
        idx_hbm.at[base // 128, pl.ds(base % 128, _TOK_PER_W)], idx_v)
    vbufs = (vrows0, vrows1)
    kbufs = (krows0, krows1)
    vsems = (sem_v0, sem_v1)
    ksems = (sem_k0, sem_k1)

    def start(c):
        sl = idx_v.at[pl.ds(c * _SC_CHUNK, _SC_CHUNK)]
        cv = pltpu.async_copy(vtab_hbm.at[sl], vbufs[c % 2], vsems[c % 2])
        ck = pltpu.async_copy(ktab_hbm.at[sl], kbufs[c % 2], ksems[c % 2])
        return cv, ck

    pend = start(0)
    for c in range(_SC_STEPS):
        cv, ck = pend
        if c + 1 < _SC_STEPS:
            nxt = start(c + 1)
        cv.wait()
        ck.wait()
        if c + 1 < _SC_STEPS:
            pend = nxt
        off = base + c * _SC_CHUNK
        pltpu.sync_copy(vbufs[c % 2], pred_hbm.at[pl.ds(off, _SC_CHUNK)])
        pltpu.sync_copy(kbufs[c % 2], ksel_hbm.at[pl.ds(off, _SC_CHUNK)])


def kernel(h_in, times, Wk, bk, Wp, bp, W1, b1, W2, b2, g_in, b_in, g_pr,
           b_pr, proj, K_mem, V_mem):
    f32 = jnp.float32
    h2 = h_in.reshape(_N, _D_IN)
    tcol = times.reshape(_N, 1).astype(f32)

    n_head = _PN // _HEAD_TB
    rows_per_step = _HEAD_TB // 128
    head_w = (Wk, bk, Wp, Wp, bp, proj, jnp.asarray(_pf45_table()))

    def run_head(part):
        off = part * n_head
        return pl.pallas_call(
            _head_body,
            grid=(n_head,),
            in_specs=[
                pl.BlockSpec((_HEAD_TB, _D_IN), lambda i, o=off: (i + o, 0)),
                pl.BlockSpec((_HEAD_TB, 1), lambda i, o=off: (i + o, 0)),
                pl.BlockSpec((_D_IN, _D_KEY), lambda i: (0, 0)),
                pl.BlockSpec((_D_KEY,), lambda i: (0,)),
                pl.BlockSpec((_D_KEY, _D_KEY), lambda i: (0, 0)),
                pl.BlockSpec((8, _D_KEY), lambda i: (_D_KEY // 8, 0)),
                pl.BlockSpec((_D_KEY,), lambda i: (0,)),
                pl.BlockSpec((_D_KEY, _N_BITS), lambda i: (0, 0)),
                pl.BlockSpec((45, 8), lambda i: (0, 0)),
            ],
            out_specs=[
                pl.BlockSpec((_HEAD_TB, _D_KEY), lambda i: (i, 0)),
                pl.BlockSpec((rows_per_step, 128), lambda i: (i, 0)),
            ],
            out_shape=[
                jax.ShapeDtypeStruct((_PN, _D_KEY), f32),
                jax.ShapeDtypeStruct((_PN // 128, 128), jnp.int32),
            ],
        )(h2, tcol, *head_w)

    mesh = plsc.VectorSubcoreMesh(core_axis_name="c", subcore_axis_name="s")
    gather = pl.kernel(
        _sc_gather_body,
        out_type=(
            jax.ShapeDtypeStruct((_PN, _D_VAL), f32),
            jax.ShapeDtypeStruct((_PN, _D_KEY), f32),
        ),
        mesh=mesh,
        scratch_types=[
            pltpu.VMEM((_TOK_PER_W,), jnp.int32),
            pltpu.VMEM((_SC_CHUNK, _D_VAL), f32),
            pltpu.VMEM((_SC_CHUNK, _D_VAL), f32),
            pltpu.VMEM((_SC_CHUNK, _D_KEY), f32),
            pltpu.VMEM((_SC_CHUNK, _D_KEY), f32),
            pltpu.SemaphoreType.DMA,
            pltpu.SemaphoreType.DMA,
            pltpu.SemaphoreType.DMA,
            pltpu.SemaphoreType.DMA,
        ],
    )

    n_tail = _PN // _TAIL_TB
    tail_w = (W1, W1, W1, b1, W2, b2, g_in, b_in, g_pr, b_pr)
    tail_w_specs = [
        pl.BlockSpec((_D_IN, _D_IN), lambda i: (0, 0)),
        pl.BlockSpec((_D_VAL, _D_IN), lambda i: (1, 0)),
        pl.BlockSpec((8, _D_IN), lambda i: ((_D_IN + _D_VAL) // 8, 0)),
        pl.BlockSpec((_D_IN,), lambda i: (0,)),
        pl.BlockSpec((_D_IN, 1), lambda i: (0, 0)),
        pl.BlockSpec((1,), lambda i: (0,)),
        pl.BlockSpec((_D_IN,), lambda i: (0,)),
        pl.BlockSpec((_D_IN,), lambda i: (0,)),
        pl.BlockSpec((_D_VAL,), lambda i: (0,)),
        pl.BlockSpec((_D_VAL,), lambda i: (0,)),
    ]

    def run_tail(part, pred, ksel, keys_p, y_prev):
        off = part * n_tail
        in_specs = [
            pl.BlockSpec((_TAIL_TB, _D_IN), lambda i, o=off: (i + o, 0)),
            pl.BlockSpec((_TAIL_TB, _D_VAL), lambda i: (i, 0)),
            pl.BlockSpec((_TAIL_TB, _D_KEY), lambda i: (i, 0)),
            pl.BlockSpec((_TAIL_TB, _D_KEY), lambda i: (i, 0)),
        ] + list(tail_w_specs)
        operands = [h2, pred, keys_p, ksel, *tail_w]
        body = _tail_body
        aliases = {}
        if y_prev is not None:
            body = _tail_body_alias
            in_specs.append(pl.BlockSpec((8, 128), lambda i: (0, 0)))
            operands.append(y_prev)
            aliases = {len(operands) - 1: 0}
        return pl.pallas_call(
            body,
            grid=(n_tail,),
            in_specs=in_specs,
            out_specs=[
                pl.BlockSpec((_TAIL_TB, _D_IN), lambda i, o=off: (i + o, 0)),
                pl.BlockSpec((1, 1), lambda i: (0, 0)),
                pl.BlockSpec((1, 1), lambda i: (0, 0)),
            ],
            out_shape=[
                jax.ShapeDtypeStruct((_N, _D_IN), f32),
                jax.ShapeDtypeStruct((1, 1), f32),
                jax.ShapeDtypeStruct((1, 1), f32),
            ],
            input_output_aliases=aliases,
        )(*operands)

    heads = [run_head(p) for p in range(_PARTS)]
    gathers = [gather(idx_p, V_mem, K_mem) for _, idx_p in heads]
    y = None
    asums, csums = [], []
    for p in range(_PARTS):
        keys_p = heads[p][0]
        pred_p, ksel_p = gathers[p]
        y, a_p, c_p = run_tail(p, pred_p, ksel_p, keys_p, y)
        asums.append(a_p[0, 0])
        csums.append(c_p[0, 0])

    y_out = y.reshape(_B, _L, _D_IN)
    inv_n = jnp.float32(1.0 / _N)
    return (y_out, sum(asums) * inv_n, sum(csums) * inv_n)


# speedup vs baseline: 1.1590x; 1.1590x over previous
import functools

import jax
import jax.numpy as jnp
import numpy as np
from jax import lax
from jax.experimental import pallas as pl
from jax.experimental.pallas import tpu as pltpu
from jax.experimental.pallas import tpu_sc as plsc

_B, _L, _D_IN = 4, 2048, 768
_D_KEY, _D_VAL = 128, 768
_N_BITS = 16
_N = _B * _L

_SC_NC = 2
_SC_NS = 16
_SC_NW = _SC_NC * _SC_NS
_PARTS = 2
_PN = _N // _PARTS
_TOK_PER_W = _PN // _SC_NW
_SC_CHUNK = 64
_SC_STEPS = _TOK_PER_W // _SC_CHUNK

_HEAD_TB = 1024
_TAIL_TB = 512


def _pf45_table() -> np.ndarray:
    r = np.arange(45, dtype=np.float64)
    a = 2.0 * np.pi * r
    cols = np.stack([
        np.cos(a), np.cos(a / 3.0), np.cos(a / 9.0),
        np.sin(a), np.sin(a / 3.0), np.sin(a / 9.0),
        (r % 5 == 0).astype(np.float64), (r % 5 == 1).astype(np.float64),
    ], axis=1)
    return np.tanh(cols).astype(np.float32)


def _head_body(h_ref, t_ref, wk_ref, bk_ref, wpk_ref, wpp_ref, bp_ref,
               proj_ref, pf45_ref, keys_ref, idx_ref):
    h = h_ref[...]
    t = t_ref[...]
    k1 = jnp.dot(h, wk_ref[...], preferred_element_type=jnp.float32) + bk_ref[...]
    r45 = t - 45.0 * jnp.floor(t / 45.0)
    oh = (lax.broadcasted_iota(jnp.int32, (_HEAD_TB, 45), 1)
          == r45.astype(jnp.int32)).astype(jnp.float32)
    tbl = jnp.dot(pf45_ref[...], wpp_ref[...],
                  preferred_element_type=jnp.float32)
    keys = (jnp.dot(k1, wpk_ref[...], preferred_element_type=jnp.float32)
            + jnp.dot(oh, tbl, preferred_element_type=jnp.float32)
            + bp_ref[...])
    keys_ref[...] = keys
    logits = jnp.dot(keys, proj_ref[...], preferred_element_type=jnp.float32)
    w = (jnp.int32(1) << jnp.arange(_N_BITS, dtype=jnp.int32))[None, :]
    idx = jnp.sum((logits > 0.0).astype(jnp.int32) * w, axis=1, keepdims=True)
    idx_ref[...] = idx.reshape(_HEAD_TB // 128, 128)


def _tail_body(h_ref, p_ref, keys_ref, ksel_ref, w1a_ref, w1b_ref, w1c_ref,
               b1_ref, w2_ref, b2_ref, gin_ref, bin_ref, gpr_ref, bpr_ref,
               y_ref, asum_ref, csum_ref):
    def ln(x, g, b):
        m = jnp.mean(x, axis=1, keepdims=True)
        v = jnp.mean((x - m) ** 2, axis=1, keepdims=True)
        return (x - m) / jnp.sqrt(v + 1e-5) * g + b

    h = h_ref[...]
    p = p_ref[...]
    lnh = ln(h, gin_ref[...], bin_ref[...])
    lnp = ln(p, gpr_ref[...], bpr_ref[...])
    conf = jax.nn.sigmoid(
        jnp.sum(keys_ref[...] * ksel_ref[...], axis=1, keepdims=True)
        / jnp.sqrt(jnp.float32(_D_KEY)))
    m1 = (jnp.dot(lnh, w1a_ref[...], preferred_element_type=jnp.float32)
          + jnp.dot(lnp, w1b_ref[...], preferred_element_type=jnp.float32)
          + conf * w1c_ref[0:1, :] + b1_ref[...])
    s = m1 * jax.nn.sigmoid(m1)
    pre = jnp.dot(s, w2_ref[...], preferred_element_type=jnp.float32) + b2_ref[...]
    alpha = jnp.clip(jax.nn.sigmoid(pre), 0.0, 1.0)
    y_ref[...] = (1.0 - alpha) * h + alpha * (h + p)

    @pl.when(pl.program_id(0) == 0)
    def _():
        asum_ref[...] = jnp.zeros_like(asum_ref)
        csum_ref[...] = jnp.zeros_like(csum_ref)

    asum_ref[...] += jnp.sum(alpha).reshape(1, 1)
    csum_ref[...] += jnp.sum(conf).reshape(1, 1)


def _tail_body_alias(h_ref, p_ref, keys_ref, ksel_ref, w1a_ref, w1b_ref,
                     w1c_ref, b1_ref, w2_ref, b2_ref, gin_ref, bin_ref,
                     gpr_ref, bpr_ref, y0_ref, y_ref, asum_ref, csum_ref):
    del y0_ref
    _tail_body(h_ref, p_ref, keys_ref, ksel_ref, w1a_ref, w1b_ref, w1c_ref,
               b1_ref, w2_ref, b2_ref, gin_ref, bin_ref, gpr_ref, bpr_ref,
               y_ref, asum_ref, csum_ref)


def _sc_gather_body(idx_hbm, vtab_hbm, ktab_hbm, pred_hbm, ksel_hbm,
                    idx_v, vrows0, vrows1, krows0, krows1,
                    sem_v0, sem_v1, sem_k0, sem_k1, sem_w0, sem_w1):
    wid = lax.axis_index("s") * _SC_NC + lax.axis_index("c")
    base = wid * _TOK_PER_W
    pltpu.sync_copy(
        idx_hbm.at[base // 128, pl.ds(base % 128, _TOK_PER_W)], idx_v)
    vbufs = (vrows0, vrows1)
    kbufs = (krows0, krows1)
    vsems = (sem_v0, sem_v1)
    ksems = (sem_k0, sem_k1)
    wsems = (sem_w0, sem_w1)

    gathers = []
    for c in range(_SC_STEPS):
        sl = idx_v.at[pl.ds(c * _SC_CHUNK, _SC_CHUNK)]
        cv = pltpu.async_copy(vtab_hbm.at[sl], vbufs[c], vsems[c])
        ck = pltpu.async_copy(ktab_hbm.at[sl], kbufs[c], ksems[c])
        gathers.append((cv, ck))
    writes = []
    for c in range(_SC_STEPS):
        cv, ck = gathers[c]
        cv.wait()
        ck.wait()
        off = base + c * _SC_CHUNK
        writes.append(pltpu.async_copy(
            vbufs[c], pred_hbm.at[pl.ds(off, _SC_CHUNK)], wsems[c]))
        writes.append(pltpu.async_copy(
            kbufs[c], ksel_hbm.at[pl.ds(off, _SC_CHUNK)], wsems[c]))
    for w in writes:
        w.wait()


def kernel(h_in, times, Wk, bk, Wp, bp, W1, b1, W2, b2, g_in, b_in, g_pr,
           b_pr, proj, K_mem, V_mem):
    f32 = jnp.float32
    h2 = h_in.reshape(_N, _D_IN)
    tcol = times.reshape(_N, 1).astype(f32)

    n_head = _PN // _HEAD_TB
    rows_per_step = _HEAD_TB // 128
    head_w = (Wk, bk, Wp, Wp, bp, proj, jnp.asarray(_pf45_table()))

    def run_head(part):
        off = part * n_head
        return pl.pallas_call(
            _head_body,
            grid=(n_head,),
            in_specs=[
                pl.BlockSpec((_HEAD_TB, _D_IN), lambda i, o=off: (i + o, 0)),
                pl.BlockSpec((_HEAD_TB, 1), lambda i, o=off: (i + o, 0)),
                pl.BlockSpec((_D_IN, _D_KEY), lambda i: (0, 0)),
                pl.BlockSpec((_D_KEY,), lambda i: (0,)),
                pl.BlockSpec((_D_KEY, _D_KEY), lambda i: (0, 0)),
                pl.BlockSpec((8, _D_KEY), lambda i: (_D_KEY // 8, 0)),
                pl.BlockSpec((_D_KEY,), lambda i: (0,)),
                pl.BlockSpec((_D_KEY, _N_BITS), lambda i: (0, 0)),
                pl.BlockSpec((45, 8), lambda i: (0, 0)),
            ],
            out_specs=[
                pl.BlockSpec((_HEAD_TB, _D_KEY), lambda i: (i, 0)),
                pl.BlockSpec((rows_per_step, 128), lambda i: (i, 0)),
            ],
            out_shape=[
                jax.ShapeDtypeStruct((_PN, _D_KEY), f32),
                jax.ShapeDtypeStruct((_PN // 128, 128), jnp.int32),
            ],
        )(h2, tcol, *head_w)

    mesh = plsc.VectorSubcoreMesh(core_axis_name="c", subcore_axis_name="s")
    gather = pl.kernel(
        _sc_gather_body,
        out_type=(
            jax.ShapeDtypeStruct((_PN, _D_VAL), f32),
            jax.ShapeDtypeStruct((_PN, _D_KEY), f32),
        ),
        mesh=mesh,
        scratch_types=[
            pltpu.VMEM((_TOK_PER_W,), jnp.int32),
            pltpu.VMEM((_SC_CHUNK, _D_VAL), f32),
            pltpu.VMEM((_SC_CHUNK, _D_VAL), f32),
            pltpu.VMEM((_SC_CHUNK, _D_KEY), f32),
            pltpu.VMEM((_SC_CHUNK, _D_KEY), f32),
            pltpu.SemaphoreType.DMA,
            pltpu.SemaphoreType.DMA,
            pltpu.SemaphoreType.DMA,
            pltpu.SemaphoreType.DMA,
            pltpu.SemaphoreType.DMA,
            pltpu.SemaphoreType.DMA,
        ],
    )

    n_tail = _PN // _TAIL_TB
    tail_w = (W1, W1, W1, b1, W2, b2, g_in, b_in, g_pr, b_pr)
    tail_w_specs = [
        pl.BlockSpec((_D_IN, _D_IN), lambda i: (0, 0)),
        pl.BlockSpec((_D_VAL, _D_IN), lambda i: (1, 0)),
        pl.BlockSpec((8, _D_IN), lambda i: ((_D_IN + _D_VAL) // 8, 0)),
        pl.BlockSpec((_D_IN,), lambda i: (0,)),
        pl.BlockSpec((_D_IN, 1), lambda i: (0, 0)),
        pl.BlockSpec((1,), lambda i: (0,)),
        pl.BlockSpec((_D_IN,), lambda i: (0,)),
        pl.BlockSpec((_D_IN,), lambda i: (0,)),
        pl.BlockSpec((_D_VAL,), lambda i: (0,)),
        pl.BlockSpec((_D_VAL,), lambda i: (0,)),
    ]

    def run_tail(part, pred, ksel, keys_p, y_prev):
        off = part * n_tail
        in_specs = [
            pl.BlockSpec((_TAIL_TB, _D_IN), lambda i, o=off: (i + o, 0)),
            pl.BlockSpec((_TAIL_TB, _D_VAL), lambda i: (i, 0)),
            pl.BlockSpec((_TAIL_TB, _D_KEY), lambda i: (i, 0)),
            pl.BlockSpec((_TAIL_TB, _D_KEY), lambda i: (i, 0)),
        ] + list(tail_w_specs)
        operands = [h2, pred, keys_p, ksel, *tail_w]
        body = _tail_body
        aliases = {}
        if y_prev is not None:
            body = _tail_body_alias
            in_specs.append(pl.BlockSpec((8, 128), lambda i: (0, 0)))
            operands.append(y_prev)
            aliases = {len(operands) - 1: 0}
        return pl.pallas_call(
            body,
            grid=(n_tail,),
            in_specs=in_specs,
            out_specs=[
                pl.BlockSpec((_TAIL_TB, _D_IN), lambda i, o=off: (i + o, 0)),
                pl.BlockSpec((1, 1), lambda i: (0, 0)),
                pl.BlockSpec((1, 1), lambda i: (0, 0)),
            ],
            out_shape=[
                jax.ShapeDtypeStruct((_N, _D_IN), f32),
                jax.ShapeDtypeStruct((1, 1), f32),
                jax.ShapeDtypeStruct((1, 1), f32),
            ],
            input_output_aliases=aliases,
        )(*operands)

    heads = [run_head(p) for p in range(_PARTS)]
    gathers = [gather(idx_p, V_mem, K_mem) for _, idx_p in heads]
    y = None
    asums, csums = [], []
    for p in range(_PARTS):
        keys_p = heads[p][0]
        pred_p, ksel_p = gathers[p]
        y, a_p, c_p = run_tail(p, pred_p, ksel_p, keys_p, y)
        asums.append(a_p[0, 0])
        csums.append(c_p[0, 0])

    y_out = y.reshape(_B, _L, _D_IN)
    inv_n = jnp.float32(1.0 / _N)
    return (y_out, sum(asums) * inv_n, sum(csums) * inv_n)
